# baseline (speedup 1.0000x reference)
import jax
import jax.numpy as jnp
from jax.experimental import pallas as pl


def _mlp_head_kernel(x_ref, w1_ref, b1_ref, w2_ref, b2_ref, o_ref):
    i = pl.program_id(0)
    xf = x_ref[...]
    h = jax.lax.dot_general(
        xf, w1_ref[...],
        dimension_numbers=(((1,), (1,)), ((), ())),
        preferred_element_type=jnp.float32,
    )
    h = jnp.maximum(h + b1_ref[0], 0.0)
    part = jnp.sum(h * w2_ref[0], keepdims=True)

    @pl.when(i == 0)
    def _():
        o_ref[...] = part + b2_ref[...]

    @pl.when(i != 0)
    def _():
        o_ref[...] += part


def kernel(x, edge_index, conv1_w_rel, conv1_b_rel, conv1_w_root,
           conv2_w_rel, conv2_b_rel, conv2_w_root,
           fc1_w, fc1_b, fc2_w, fc2_b):
    B = 144
    out = pl.pallas_call(
        _mlp_head_kernel,
        grid=(2,),
        in_specs=[
            pl.BlockSpec((1, 288), lambda i: (0, 0)),
            pl.BlockSpec((B, 288), lambda i: (i, 0)),
            pl.BlockSpec((1, 1, B), lambda i: (i, 0, 0)),
            pl.BlockSpec((1, 1, B), lambda i: (i, 0, 0)),
            pl.BlockSpec((1, 1), lambda i: (0, 0)),
        ],
        out_specs=pl.BlockSpec((1, 1), lambda i: (0, 0)),
        out_shape=jax.ShapeDtypeStruct((1, 1), jnp.float32),
    )(x.reshape(1, 288), fc1_w, fc1_b.reshape(2, 1, B), fc2_w.reshape(2, 1, B),
      fc2_b.reshape(1, 1))
    return out.reshape(1)


# speedup vs baseline: 1.2020x; 1.2020x over previous
import jax
import jax.numpy as jnp
from jax.experimental import pallas as pl
from jax.experimental.pallas import tpu as pltpu


def _mlp_head_kernel(b2_ref, x_ref, w1_ref, b1_ref, w2_ref, o_ref):
    xf = x_ref[...]
    h = jax.lax.dot_general(
        xf, w1_ref[...],
        dimension_numbers=(((1,), (1,)), ((), ())),
        preferred_element_type=jnp.float32,
    )
    h = jnp.maximum(h + b1_ref[...], 0.0)
    o_ref[...] = jnp.sum(h * w2_ref[...], keepdims=True) + b2_ref[0]


def kernel(x, edge_index, conv1_w_rel, conv1_b_rel, conv1_w_root,
           conv2_w_rel, conv2_b_rel, conv2_w_root,
           fc1_w, fc1_b, fc2_w, fc2_b):
    out = pl.pallas_call(
        _mlp_head_kernel,
        in_specs=[
            pl.BlockSpec(memory_space=pltpu.SMEM),
            pl.BlockSpec(memory_space=pltpu.VMEM),
            pl.BlockSpec(memory_space=pltpu.VMEM),
            pl.BlockSpec(memory_space=pltpu.VMEM),
            pl.BlockSpec(memory_space=pltpu.VMEM),
        ],
        out_shape=jax.ShapeDtypeStruct((1, 1), jnp.float32),
    )(fc2_b, x.reshape(1, 288), fc1_w, fc1_b.reshape(1, 288), fc2_w)
    return out.reshape(1)
